# baseline (speedup 1.0000x reference)
import functools

import jax
import jax.numpy as jnp
from jax import lax
from jax.experimental import pallas as pl
from jax.experimental.pallas import tpu as pltpu
from jax.experimental.pallas import tpu_sc as plsc

B, S, D = 4, 2048, 2048
HALF = D // 2
NUM_SPANS = 2048
WIDTH_DIM = 64
OUT_D = 2 * HALF + WIDTH_DIM

NW = 32
P = (B * NUM_SPANS) // NW
C = 8
NCHUNK = P // C
WPB = NW // B


def _sc_body(seq2, starts, ends, sent_s_h, sent_e_h, wemb_h, out,
             fs0, fe0, bx0, bs0, fs1, fe1, bx1, bs1,
             wc0, wc1, wemb_buf,
             sent_s, sent_e, st_ref, en_ref,
             fs_idx, fe_idx, bx_idx, bs_idx, ms_ref, me_ref, wb_idx,
             sem_g0, sem_g1, sem_wr0, sem_wr1, sem_w):
    c_id = lax.axis_index("c")
    s_id = lax.axis_index("s")
    wid = s_id * 2 + c_id
    r0 = pl.multiple_of(wid * P, P)
    baseS = jnp.full((16,), (wid // WPB) * S, jnp.int32)

    gsets = ((fs0, fe0, bx0, bs0, sem_g0), (fs1, fe1, bx1, bs1, sem_g1))
    wsets = ((wc0, sem_wr0), (wc1, sem_wr1))

    pltpu.sync_copy(starts.at[pl.ds(r0, P)], st_ref)
    pltpu.sync_copy(ends.at[pl.ds(r0, P)], en_ref)
    pltpu.sync_copy(sent_s_h, sent_s)
    pltpu.sync_copy(sent_e_h, sent_e)

    for i in range(P // 16):
        sl = pl.ds(i * 16, 16)
        s_v = st_ref[sl]
        e_v = en_ref[sl]
        fs_idx[sl] = baseS + jnp.maximum(s_v - 1, 0)
        fe_idx[sl] = baseS + e_v
        bx_idx[sl] = baseS + jnp.minimum(e_v + 1, S - 1)
        bs_idx[sl] = baseS + s_v
        ms_ref[sl] = (1 - jnp.minimum(s_v, 1)).astype(jnp.float32)
        me_ref[sl] = (1 - jnp.minimum((S - 1) - e_v, 1)).astype(jnp.float32)
        w_v = e_v - s_v
        one = jnp.int32(1)
        zero = jnp.int32(0)
        bkt = (jnp.minimum(w_v, 4)
               + jnp.minimum(jnp.maximum(w_v - 4, zero), one)
               + jnp.minimum(jnp.maximum(w_v - 7, zero), one)
               + jnp.minimum(jnp.maximum(w_v - 15, zero), one)
               + jnp.minimum(jnp.maximum(w_v - 31, zero), one)
               + jnp.minimum(jnp.maximum(w_v - 63, zero), one))
        wb_idx[i // 8, pl.ds((i % 8) * 16, 16)] = bkt

    def gather_pairs(c, par):
        fsb, feb, bxb, bsb, sem = gsets[par]
        isl = pl.ds(pl.multiple_of(c * C, C), C)
        fsl = pl.ds(0, HALF)
        bsl = pl.ds(HALF, HALF)
        return ((seq2.at[fs_idx.at[isl], fsl], fsb, sem),
                (seq2.at[fe_idx.at[isl], fsl], feb, sem),
                (seq2.at[bx_idx.at[isl], bsl], bxb, sem),
                (seq2.at[bs_idx.at[isl], bsl], bsb, sem))

    def fire_gathers(c, par):
        for src, dst, sem in gather_pairs(c, par):
            pltpu.async_copy(src, dst, sem)

    def wait_gathers(c, par):
        for src, dst, sem in gather_pairs(c, par):
            pltpu.make_async_copy(src, dst, sem).wait()

    def out_slab(c, par):
        wcb, sem = wsets[par]
        rows = pl.ds(r0 + pl.multiple_of(c * C, C), C)
        return wcb, out.at[rows, pl.ds(0, 2 * HALF)], sem

    def compute_chunk(c, par):
        fsb, feb, bxb, bsb, _ = gsets[par]
        wcb, _ = wsets[par]
        off = pl.multiple_of(c * C, C)
        ms_chunk = ms_ref[pl.ds(off, 16)]
        me_chunk = me_ref[pl.ds(off, 16)]

        def span_body(i, carry2):
            gvec = jnp.full((16,), i, jnp.int32)
            ms = ms_chunk.at[gvec].get(mode="promise_in_bounds")
            me = me_chunk.at[gvec].get(mode="promise_in_bounds")
            norm = (ms + me) + 0.0 * lax.iota(jnp.int32, 16).astype(jnp.float32)
            has_sent = norm[0] > 0.0

            def slow_path():
                def dim_body(j, carry3):
                    jo = pl.multiple_of(j * 16, 16)
                    dsl = pl.ds(jo, 16)
                    bsl = pl.ds(HALF + jo, 16)
                    wcb[i, dsl] = (feb[i, dsl] - fsb[i, dsl] * (1.0 - ms)
                                   - sent_s[dsl] * ms)
                    wcb[i, bsl] = (bxb[i, dsl] * (1.0 - me)
                                   + sent_e[dsl] * me - bsb[i, dsl])
                    return carry3

                lax.fori_loop(0, HALF // 16, dim_body, 0)

            def fast_path():
                for j in range(HALF // 16):
                    dsl = pl.ds(j * 16, 16)
                    bsl = pl.ds(HALF + j * 16, 16)
                    wcb[i, dsl] = feb[i, dsl] - fsb[i, dsl]
                    wcb[i, bsl] = bxb[i, dsl] - bsb[i, dsl]

            lax.cond(has_sent, slow_path, fast_path)
            return carry2

        lax.fori_loop(0, C, span_body, 0)

    fire_gathers(0, 0)
    pltpu.async_copy(wemb_h.at[wb_idx.at[0]], wemb_buf, sem_w)

    def pair_body(g, carry):
        for par in (0, 1):
            c = g * 2 + par

            @pl.when(c + 1 < NCHUNK)
            def _():
                fire_gathers(c + 1, 1 - par)

            @pl.when(c >= 2)
            def _():
                src, dst, sem = out_slab(c - 2, par)
                pltpu.make_async_copy(src, dst, sem).wait()

            wait_gathers(c, par)
            compute_chunk(c, par)
            src, dst, sem = out_slab(c, par)
            pltpu.async_copy(src, dst, sem)
        return carry

    lax.fori_loop(0, NCHUNK // 2, pair_body, 0)
    for c, par in ((NCHUNK - 2, 0), (NCHUNK - 1, 1)):
        src, dst, sem = out_slab(c, par)
        pltpu.make_async_copy(src, dst, sem).wait()

    for h in range(2):
        hbase = pl.multiple_of(r0 + h * 128, 128)
        pltpu.make_async_copy(wemb_h.at[wb_idx.at[h]], wemb_buf, sem_w).wait()

        def wrow(r, carry):
            src = wemb_buf.at[r, pl.ds(0, WIDTH_DIM)]
            dst = out.at[hbase + r, pl.ds(2 * HALF, WIDTH_DIM)]
            pltpu.async_copy(src, dst, sem_w)
            return carry

        lax.fori_loop(0, 128, wrow, 0)

        def wrow_wait(r, carry):
            src = wemb_buf.at[r, pl.ds(0, WIDTH_DIM)]
            dst = out.at[hbase + r, pl.ds(2 * HALF, WIDTH_DIM)]
            pltpu.make_async_copy(src, dst, sem_w).wait()
            return carry

        lax.fori_loop(0, 128, wrow_wait, 0)

        if h == 0:
            pltpu.async_copy(wemb_h.at[wb_idx.at[1]], wemb_buf, sem_w)


_sc_call = functools.partial(
    pl.kernel,
    mesh=plsc.VectorSubcoreMesh(core_axis_name="c", subcore_axis_name="s"),
    out_type=jax.ShapeDtypeStruct((B * NUM_SPANS, OUT_D), jnp.float32),
    scratch_types=[
        pltpu.VMEM((C, HALF), jnp.float32),
        pltpu.VMEM((C, HALF), jnp.float32),
        pltpu.VMEM((C, HALF), jnp.float32),
        pltpu.VMEM((C, HALF), jnp.float32),
        pltpu.VMEM((C, HALF), jnp.float32),
        pltpu.VMEM((C, HALF), jnp.float32),
        pltpu.VMEM((C, HALF), jnp.float32),
        pltpu.VMEM((C, HALF), jnp.float32),
        pltpu.VMEM((C, 2 * HALF), jnp.float32),
        pltpu.VMEM((C, 2 * HALF), jnp.float32),
        pltpu.VMEM((128, 128), jnp.float32),
        pltpu.VMEM((HALF,), jnp.float32),
        pltpu.VMEM((HALF,), jnp.float32),
        pltpu.VMEM((P,), jnp.int32),
        pltpu.VMEM((P,), jnp.int32),
        pltpu.VMEM((P,), jnp.int32),
        pltpu.VMEM((P,), jnp.int32),
        pltpu.VMEM((P,), jnp.int32),
        pltpu.VMEM((P,), jnp.int32),
        pltpu.VMEM((P + 16,), jnp.float32),
        pltpu.VMEM((P + 16,), jnp.float32),
        pltpu.VMEM((2, 128), jnp.int32),
        pltpu.SemaphoreType.DMA,
        pltpu.SemaphoreType.DMA,
        pltpu.SemaphoreType.DMA,
        pltpu.SemaphoreType.DMA,
        pltpu.SemaphoreType.DMA,
    ],
)(_sc_body)


def kernel(sequence_tensor, span_indices, start_sentinel, end_sentinel,
           width_embedding):
    seq2 = sequence_tensor.reshape(B * S, D)
    starts = span_indices[..., 0].reshape(-1).astype(jnp.int32)
    ends = span_indices[..., 1].reshape(-1).astype(jnp.int32)
    wemb_p = jnp.zeros((width_embedding.shape[0], 128),
                       width_embedding.dtype).at[:, :WIDTH_DIM].set(width_embedding)
    out = _sc_call(seq2, starts, ends,
                   start_sentinel.reshape(HALF),
                   end_sentinel.reshape(HALF),
                   wemb_p)
    return out.reshape(B, NUM_SPANS, OUT_D)
